# initial kernel scaffold (unmeasured)
import jax
import jax.numpy as jnp
from jax import lax
from jax.experimental import pallas as pl
from jax.experimental.pallas import tpu as pltpu

N_DEV = 8


def kernel(x, w_mat, scale_x, scale_w):
    m_per, k = x.shape
    _, n = w_mat.shape
    n_per = n // N_DEV

    def body(x_ref, w_hbm, sx_ref, sw_ref, out_ref,
             xg_ref, w_col, w_bf, send_sems, recv_sems, wdma_sem):
        my = lax.axis_index("i")

        wcopy = pltpu.make_async_copy(
            w_hbm.at[:, pl.ds(my * n_per, n_per)], w_col, wdma_sem)
        wcopy.start()

        bsem = pltpu.get_barrier_semaphore()
        for h in range(1, N_DEV):
            dst = lax.rem(my + h, N_DEV)
            pl.semaphore_signal(bsem, inc=1, device_id=(dst,),
                                device_id_type=pl.DeviceIdType.MESH)
        pl.semaphore_wait(bsem, N_DEV - 1)

        sends = []
        for h in range(1, N_DEV):
            dst = lax.rem(my + h, N_DEV)
            rdma = pltpu.make_async_remote_copy(
                src_ref=x_ref,
                dst_ref=xg_ref.at[my],
                send_sem=send_sems.at[h - 1],
                recv_sem=recv_sems.at[my],
                device_id=(dst,),
                device_id_type=pl.DeviceIdType.MESH,
            )
            rdma.start()
            sends.append(rdma)

        wcopy.wait()
        w_bf[...] = w_col[...].astype(jnp.bfloat16)
        scale = sx_ref[0] * sw_ref[0]

        own = jnp.dot(x_ref[...].astype(jnp.bfloat16), w_bf[...],
                      preferred_element_type=jnp.float32)
        out_ref[pl.ds(my * m_per, m_per), :] = own * scale

        for h in range(1, N_DEV):
            src = lax.rem(my - h + N_DEV, N_DEV)
            recv = pltpu.make_async_remote_copy(
                src_ref=x_ref,
                dst_ref=xg_ref.at[src],
                send_sem=send_sems.at[0],
                recv_sem=recv_sems.at[src],
                device_id=(src,),
                device_id_type=pl.DeviceIdType.MESH,
            )
            recv.wait_recv()
            blk = jnp.dot(xg_ref[src].astype(jnp.bfloat16), w_bf[...],
                          preferred_element_type=jnp.float32)
            out_ref[pl.ds(src * m_per, m_per), :] = blk * scale

        for rdma in sends:
            rdma.wait_send()

    out_shape = jax.ShapeDtypeStruct((N_DEV * m_per, n_per), jnp.float32)
    return pl.pallas_call(
        body,
        out_shape=out_shape,
        in_specs=[
            pl.BlockSpec(memory_space=pltpu.VMEM),
            pl.BlockSpec(memory_space=pltpu.ANY),
            pl.BlockSpec(memory_space=pltpu.SMEM),
            pl.BlockSpec(memory_space=pltpu.SMEM),
        ],
        out_specs=pl.BlockSpec(memory_space=pltpu.VMEM),
        scratch_shapes=[
            pltpu.VMEM((N_DEV, m_per, k), jnp.int8),
            pltpu.VMEM((k, n_per), jnp.int8),
            pltpu.VMEM((k, n_per), jnp.bfloat16),
            pltpu.SemaphoreType.DMA((N_DEV - 1,)),
            pltpu.SemaphoreType.DMA((N_DEV,)),
            pltpu.SemaphoreType.DMA,
        ],
        compiler_params=pltpu.CompilerParams(collective_id=0),
    )(x, w_mat, scale_x, scale_w)


# baseline (device time: 155576 ns/iter reference)
import jax
import jax.numpy as jnp
from jax import lax
from jax.experimental import pallas as pl
from jax.experimental.pallas import tpu as pltpu

N_DEV = 8


def kernel(x, w_mat, scale_x, scale_w):
    m_per, k = x.shape
    _, n = w_mat.shape
    n_per = n // N_DEV

    def body(x_ref, w_hbm, sx_ref, sw_ref, out_ref,
             xg_ref, w_col, w_bf, send_sems, recv_sems, wdma_sem):
        my = lax.axis_index("i")

        wcopy = pltpu.make_async_copy(
            w_hbm.at[:, pl.ds(my * n_per, n_per)], w_col, wdma_sem)
        wcopy.start()

        bsem = pltpu.get_barrier_semaphore()
        for h in range(1, N_DEV):
            dst = lax.rem(my + h, N_DEV)
            pl.semaphore_signal(bsem, inc=1, device_id=(dst,),
                                device_id_type=pl.DeviceIdType.MESH)
        pl.semaphore_wait(bsem, N_DEV - 1)

        sends = []
        for h in range(1, N_DEV):
            dst = lax.rem(my + h, N_DEV)
            rdma = pltpu.make_async_remote_copy(
                src_ref=x_ref,
                dst_ref=xg_ref.at[my],
                send_sem=send_sems.at[h - 1],
                recv_sem=recv_sems.at[my],
                device_id=(dst,),
                device_id_type=pl.DeviceIdType.MESH,
            )
            rdma.start()
            sends.append(rdma)

        wcopy.wait()
        w_bf[...] = w_col[...].astype(jnp.bfloat16)
        scale = sx_ref[0] * sw_ref[0]

        own = jnp.dot(x_ref[...].astype(jnp.bfloat16), w_bf[...],
                      preferred_element_type=jnp.float32)
        out_ref[pl.ds(my * m_per, m_per), :] = own * scale

        for h in range(1, N_DEV):
            src = lax.rem(my - h + N_DEV, N_DEV)
            recv = pltpu.make_async_remote_copy(
                src_ref=x_ref,
                dst_ref=xg_ref.at[src],
                send_sem=send_sems.at[0],
                recv_sem=recv_sems.at[src],
                device_id=(src,),
                device_id_type=pl.DeviceIdType.MESH,
            )
            recv.wait_recv()
            blk = jnp.dot(xg_ref[src].astype(jnp.bfloat16), w_bf[...],
                          preferred_element_type=jnp.float32)
            out_ref[pl.ds(src * m_per, m_per), :] = blk * scale

        for rdma in sends:
            rdma.wait_send()

    out_shape = jax.ShapeDtypeStruct((N_DEV * m_per, n_per), jnp.float32)
    return pl.pallas_call(
        body,
        out_shape=out_shape,
        in_specs=[
            pl.BlockSpec(memory_space=pltpu.VMEM),
            pl.BlockSpec(memory_space=pl.ANY),
            pl.BlockSpec(memory_space=pltpu.SMEM),
            pl.BlockSpec(memory_space=pltpu.SMEM),
        ],
        out_specs=pl.BlockSpec(memory_space=pltpu.VMEM),
        scratch_shapes=[
            pltpu.VMEM((N_DEV, m_per, k), jnp.int8),
            pltpu.VMEM((k, n_per), jnp.int8),
            pltpu.VMEM((k, n_per), jnp.bfloat16),
            pltpu.SemaphoreType.DMA((N_DEV - 1,)),
            pltpu.SemaphoreType.DMA((N_DEV,)),
            pltpu.SemaphoreType.DMA,
        ],
        compiler_params=pltpu.CompilerParams(collective_id=0),
    )(x, w_mat, scale_x, scale_w)
